# baseline (speedup 1.0000x reference)
import jax
import jax.numpy as jnp
from jax import lax
from jax.experimental import pallas as pl
from jax.experimental.pallas import tpu as pltpu
from jax.experimental.pallas import tpu_sc as plsc

N = 16384
D = 128
K = 819

_INT_MIN = -2147483648


def _cvec(val):
    return jnp.full((16,), val, jnp.int32)


def _to_key(f):
    b = lax.bitcast_convert_type(f, jnp.int32)
    return jnp.where(b < 0, ~b, b ^ _INT_MIN)


def _srl(x, amount):
    return lax.shift_right_logical(x, _cvec(amount))


def _scores_body(x_ref, wg_ref, bg_ref, s_ref):
    xb = x_ref[...]
    wgt = lax.transpose(wg_ref[...], (1, 0))
    col = lax.broadcasted_iota(jnp.int32, (D, D), 1)
    wgm = jnp.where(col == 0, jnp.broadcast_to(wgt, (D, D)), 0.0)
    s = jax.lax.dot_general(xb, wgm, (((1,), (0,)), ((), ())))
    s_ref[...] = (s[:, 0] + bg_ref[0, 0]).reshape(64, 128)


def _scores_tc(x, Wg, bg):
    out = pl.pallas_call(
        _scores_body,
        grid=(2,),
        in_specs=[
            pl.BlockSpec((8192, 128), lambda i: (i, 0)),
            pl.BlockSpec((1, 128), lambda i: (0, 0)),
            pl.BlockSpec(memory_space=pltpu.SMEM),
        ],
        out_specs=pl.BlockSpec((64, 128), lambda i: (i, 0)),
        out_shape=jax.ShapeDtypeStruct((128, 128), jnp.float32),
    )(x, Wg, bg.reshape(1, 1))
    return out.reshape(N)


def _select_body(scores_hbm,
                 mask_hbm,
                 sc_v, key_v, hist_v, merged_v, idxc_v, maskb_v,
                 cnt_v, cntall_v,
                 sh_merged, sh_cnt):
    c = lax.axis_index("c")
    s = lax.axis_index("s")
    j0 = 2 * s + c
    iota = lax.iota(jnp.int32, 16)
    zeros16 = jnp.zeros((16,), jnp.int32)

    pltpu.sync_copy(
        scores_hbm.at[pl.ds(pl.multiple_of(s * 1024, 1024), 1024)], sc_v)

    def kz_body(v, _):
        key_v[pl.ds(v * 16, 16)] = zeros16
        return 0
    lax.fori_loop(0, 64, kz_body, 0)

    @pl.when(s == 0)
    def _zero_merged():
        pltpu.sync_copy(key_v, sh_merged)
    plsc.subcore_barrier()

    def key_body(v, _):
        key_v[pl.ds(v * 16, 16)] = _to_key(sc_v[pl.ds(v * 16, 16)])
        return 0
    lax.fori_loop(0, 64, key_body, 0)

    rk = jnp.int32(K)
    pref = jnp.int32(0)
    for p in (3, 2, 1, 0):
        def zero_body(g, _):
            hist_v[pl.ds(g * 16, 16)] = zeros16
            return 0
        lax.fori_loop(0, 16, zero_body, 0)

        def hist_body(v, _, _p=p, _pref=pref):
            for u in (0, 1):
                kb = key_v[pl.ds((2 * v + u) * 16, 16)]
                digit = _srl(kb, 8 * _p) & _cvec(255)
                if _p == 3:
                    act = jnp.full((16,), True)
                else:
                    act = _srl(kb, 8 * _p + 8) == _pref
                cnt, last = plsc.scan_count(digit, mask=act)
                plsc.addupdate_scatter(hist_v, [digit], cnt, mask=last)
            return 0
        lax.fori_loop(0, 32, hist_body, 0)

        def idx_body(g, _, _p=p):
            idxc_v[pl.ds(g * 16, 16)] = _p * 256 + g * 16 + iota
            return 0
        lax.fori_loop(0, 16, idx_body, 0)
        pltpu.sync_copy(hist_v, sh_merged.at[idxc_v], add=True)
        plsc.subcore_barrier()
        pltpu.sync_copy(
            sh_merged.at[pl.ds(pl.multiple_of(p * 256, 256), 256)],
            merged_v)
        def suf_body(gg, carry):
            g = 15 - gg
            tot = merged_v[pl.ds(g * 16, 16)]
            suf = lax.rev(plsc.cumsum(lax.rev(tot, (0,))), (0,)) + carry
            idxc_v[pl.ds(g * 16, 16)] = suf
            return carry + jnp.sum(tot)
        lax.fori_loop(0, 16, suf_body, jnp.int32(0))

        def pick_body(g, dstar):
            idxg = iota + g * 16
            suf = idxc_v[pl.ds(g * 16, 16)]
            return jnp.maximum(dstar,
                               jnp.max(jnp.where(suf >= rk, idxg, -1)))
        dstar = lax.fori_loop(0, 16, pick_body, jnp.int32(-1))

        def cnt2_body(g, carrys):
            cntgt, cnteq = carrys
            idxg = iota + g * 16
            tot = merged_v[pl.ds(g * 16, 16)]
            cntgt = cntgt + jnp.sum(jnp.where(idxg > dstar, tot, 0))
            cnteq = cnteq + jnp.sum(jnp.where(idxg == dstar, tot, 0))
            return cntgt, cnteq
        cntgt, cnteq = lax.fori_loop(0, 16, cnt2_body,
                                     (jnp.int32(0), jnp.int32(0)))
        rk = rk - cntgt
        pref = lax.shift_left(pref, jnp.int32(8)) | dstar

    T = pref

    qinit = jnp.where(iota == 0, jnp.int32(K), 0)
    cnt_v[pl.ds(0, 16)] = qinit

    @pl.when(rk < cnteq)
    def _tie_quota():
        ties = []
        for h in (0, 1):
            def cnt_body(v, tia, _h=h):
                kb = key_v[pl.ds(_h * 512 + v * 16, 16)]
                return tia + jnp.where(kb == T, 1, 0)
            tia = lax.fori_loop(0, 32, cnt_body, zeros16)
            ties.append(jnp.sum(tia))
        cv = (jnp.where(iota == 0, ties[0], 0)
              + jnp.where(iota == 1, ties[1], 0))
        def cpad_body(q, _):
            cnt_v[pl.ds(q * 16, 16)] = zeros16
            return 0
        lax.fori_loop(1, 16, cpad_body, 0)
        cnt_v[pl.ds(0, 16)] = cv
        pltpu.sync_copy(cnt_v, sh_cnt.at[s])
        plsc.subcore_barrier()
        pltpu.sync_copy(sh_cnt, cntall_v)

        tie_acc = jnp.int32(0)
        my_quota = jnp.int32(0)
        for sp in range(16):
            row = cntall_v[sp, pl.ds(0, 16)]
            for h in range(2):
                j = 2 * sp + h
                tie_j = jnp.sum(jnp.where(iota == h, row, 0))
                quota_j = jnp.clip(rk - tie_acc, 0, tie_j)
                my_quota = jnp.where(j0 == j, quota_j, my_quota)
                tie_acc = tie_acc + tie_j
        cnt_v[pl.ds(0, 16)] = jnp.where(iota == 0, my_quota, 0)

    my_quota = jnp.sum(jnp.where(iota == 0, cnt_v[pl.ds(0, 16)], 0))

    base = j0 * 512
    Tm = T ^ _INT_MIN

    def sel_body(v, tiec):
        kb = key_v[pl.ds(c * 512 + v * 16, 16)]
        m = kb ^ _INT_MIN
        tie = kb == T
        ind = jnp.where(tie, 1, 0)
        rank = tiec + plsc.cumsum(ind) - 1
        sel = (m > Tm) | (tie & (rank < my_quota))
        maskb_v[pl.ds(v * 16, 16)] = jnp.where(sel, 1.0, 0.0)
        return tiec + jnp.sum(ind)
    lax.fori_loop(0, 32, sel_body, jnp.int32(0))

    pltpu.sync_copy(maskb_v,
                    mask_hbm.at[pl.ds(pl.multiple_of(base, 512), 512)])


def _select_sc(scores):
    mesh = plsc.VectorSubcoreMesh(core_axis_name="c", subcore_axis_name="s")
    call = pl.kernel(
        _select_body,
        out_type=jax.ShapeDtypeStruct((N,), jnp.float32),
        mesh=mesh,
        compiler_params=pltpu.CompilerParams(needs_layout_passes=False),
        scratch_types=[
            pltpu.VMEM((1024,), jnp.float32),
            pltpu.VMEM((1024,), jnp.int32),
            pltpu.VMEM((256,), jnp.int32),
            pltpu.VMEM((256,), jnp.int32),
            pltpu.VMEM((256,), jnp.int32),
            pltpu.VMEM((512,), jnp.float32),
            pltpu.VMEM((256,), jnp.int32),
            pltpu.VMEM((16, 256), jnp.int32),
            pltpu.VMEM_SHARED((1024,), jnp.int32),
            pltpu.VMEM_SHARED((16, 256), jnp.int32),
        ],
    )
    return call(scores)


def _mlp_body(x_ref, mt_ref, w1_ref, b1_ref, w2_ref, b2_ref, o_ref):
    xb = x_ref[...]
    dn = (((1,), (1,)), ((), ()))
    h = jax.lax.dot_general(xb, w1_ref[...], dn)
    h = jnp.maximum(h + b1_ref[...], 0.0)
    o = jax.lax.dot_general(h, w2_ref[...], dn) + b2_ref[...]
    mt = lax.transpose(mt_ref[...], (1, 0))
    for q in range(64):
        o_ref[128 * q:128 * (q + 1), :] = (
            o[128 * q:128 * (q + 1), :] * mt[:, q:q + 1])


def _mlp_tc(x, mask2d, W1, b1, W2, b2):
    return pl.pallas_call(
        _mlp_body,
        grid=(2,),
        in_specs=[
            pl.BlockSpec((8192, 128), lambda i: (i, 0)),
            pl.BlockSpec((64, 128), lambda i: (i, 0)),
            pl.BlockSpec((128, 128), lambda i: (0, 0)),
            pl.BlockSpec((1, 128), lambda i: (0, 0)),
            pl.BlockSpec((128, 128), lambda i: (0, 0)),
            pl.BlockSpec((1, 128), lambda i: (0, 0)),
        ],
        out_specs=pl.BlockSpec((8192, 128), lambda i: (i, 0)),
        out_shape=jax.ShapeDtypeStruct((N, D), jnp.float32),
    )(x, mask2d, W1, b1.reshape(1, D), W2, b2.reshape(1, D))


def kernel(x, W1, b1, W2, b2, Wg, bg):
    scores = _scores_tc(x, Wg, bg)
    mask = _select_sc(scores)
    out = _mlp_tc(x, mask.reshape(128, 128), W1, b1, W2, b2)
    return out, mask


# speedup vs baseline: 1.0083x; 1.0083x over previous
import jax
import jax.numpy as jnp
from jax import lax
from jax.experimental import pallas as pl
from jax.experimental.pallas import tpu as pltpu
from jax.experimental.pallas import tpu_sc as plsc

N = 16384
D = 128
K = 819

_INT_MIN = -2147483648


def _cvec(val):
    return jnp.full((16,), val, jnp.int32)


def _to_key(f):
    b = lax.bitcast_convert_type(f, jnp.int32)
    return jnp.where(b < 0, ~b, b ^ _INT_MIN)


def _srl(x, amount):
    return lax.shift_right_logical(x, _cvec(amount))


def _scores_body(x_ref, wg_ref, bg_ref, s_ref):
    xb = x_ref[...]
    wgt = lax.transpose(wg_ref[...], (1, 0))
    col = lax.broadcasted_iota(jnp.int32, (D, D), 1)
    wgm = jnp.where(col == 0, jnp.broadcast_to(wgt, (D, D)), 0.0)
    s = jax.lax.dot_general(xb, wgm, (((1,), (0,)), ((), ())))
    s_ref[...] = (s[:, 0] + bg_ref[0, 0]).reshape(64, 128)


def _scores_tc(x, Wg, bg):
    out = pl.pallas_call(
        _scores_body,
        grid=(2,),
        in_specs=[
            pl.BlockSpec((8192, 128), lambda i: (i, 0)),
            pl.BlockSpec((1, 128), lambda i: (0, 0)),
            pl.BlockSpec(memory_space=pltpu.SMEM),
        ],
        out_specs=pl.BlockSpec((64, 128), lambda i: (i, 0)),
        out_shape=jax.ShapeDtypeStruct((128, 128), jnp.float32),
    )(x, Wg, bg.reshape(1, 1))
    return out.reshape(N)


def _select_body(scores_hbm,
                 mask_hbm,
                 sc_v, key_v, hist_v, merged_v, idxc_v, maskb_v,
                 cnt_v, cntall_v,
                 sh_merged, sh_cnt):
    c = lax.axis_index("c")
    s = lax.axis_index("s")
    j0 = 2 * s + c
    iota = lax.iota(jnp.int32, 16)
    zeros16 = jnp.zeros((16,), jnp.int32)

    pltpu.sync_copy(
        scores_hbm.at[pl.ds(pl.multiple_of(s * 1024, 1024), 1024)], sc_v)

    def kz_body(v, _):
        key_v[pl.ds(v * 16, 16)] = zeros16
        return 0
    lax.fori_loop(0, 64, kz_body, 0)

    @pl.when(s == 0)
    def _zero_merged():
        pltpu.sync_copy(key_v, sh_merged)
    plsc.subcore_barrier()

    def key_body(v, _):
        key_v[pl.ds(v * 16, 16)] = _to_key(sc_v[pl.ds(v * 16, 16)])
        return 0
    lax.fori_loop(0, 64, key_body, 0)

    rk = jnp.int32(K)
    pref = jnp.int32(0)
    for p in (3, 2, 1, 0):
        def zero_body(g, _):
            hist_v[pl.ds(g * 16, 16)] = zeros16
            return 0
        lax.fori_loop(0, 16, zero_body, 0)

        def hist_body(v, _, _p=p, _pref=pref):
            for u in (0, 1):
                kb = key_v[pl.ds((2 * v + u) * 16, 16)]
                digit = _srl(kb, 8 * _p) & _cvec(255)
                if _p == 3:
                    act = jnp.full((16,), True)
                else:
                    act = _srl(kb, 8 * _p + 8) == _pref
                cnt, last = plsc.scan_count(digit, mask=act)
                plsc.addupdate_scatter(hist_v, [digit], cnt, mask=last)
            return 0
        lax.fori_loop(0, 32, hist_body, 0)

        def idx_body(g, _, _p=p):
            idxc_v[pl.ds(g * 16, 16)] = _p * 256 + g * 16 + iota
            return 0
        lax.fori_loop(0, 16, idx_body, 0)
        pltpu.sync_copy(hist_v, sh_merged.at[idxc_v], add=True)
        plsc.subcore_barrier()
        pltpu.sync_copy(
            sh_merged.at[pl.ds(pl.multiple_of(p * 256, 256), 256)],
            merged_v)
        tots = [merged_v[pl.ds(g * 16, 16)] for g in range(16)]
        sufs = [None] * 16
        carry = jnp.int32(0)
        for g in range(15, -1, -1):
            r = lax.rev(tots[g], (0,))
            sufs[g] = lax.rev(plsc.cumsum(r), (0,)) + carry
            carry = carry + jnp.sum(tots[g])
        dstar = jnp.int32(-1)
        for g in range(16):
            idxg = iota + g * 16
            dstar = jnp.maximum(
                dstar, jnp.max(jnp.where(sufs[g] >= rk, idxg, -1)))
        cntgt = jnp.int32(0)
        cnteq = jnp.int32(0)
        for g in range(16):
            idxg = iota + g * 16
            cntgt = cntgt + jnp.sum(jnp.where(idxg > dstar, tots[g], 0))
            cnteq = cnteq + jnp.sum(jnp.where(idxg == dstar, tots[g], 0))
        rk = rk - cntgt
        pref = lax.shift_left(pref, jnp.int32(8)) | dstar

    T = pref

    qinit = jnp.where(iota == 0, jnp.int32(K), 0)
    cnt_v[pl.ds(0, 16)] = qinit

    @pl.when(rk < cnteq)
    def _tie_quota():
        ties = []
        for h in (0, 1):
            def cnt_body(v, tia, _h=h):
                kb = key_v[pl.ds(_h * 512 + v * 16, 16)]
                return tia + jnp.where(kb == T, 1, 0)
            tia = lax.fori_loop(0, 32, cnt_body, zeros16)
            ties.append(jnp.sum(tia))
        cv = (jnp.where(iota == 0, ties[0], 0)
              + jnp.where(iota == 1, ties[1], 0))
        def cpad_body(q, _):
            cnt_v[pl.ds(q * 16, 16)] = zeros16
            return 0
        lax.fori_loop(1, 16, cpad_body, 0)
        cnt_v[pl.ds(0, 16)] = cv
        pltpu.sync_copy(cnt_v, sh_cnt.at[s])
        plsc.subcore_barrier()
        pltpu.sync_copy(sh_cnt, cntall_v)

        tie_acc = jnp.int32(0)
        my_quota = jnp.int32(0)
        for sp in range(16):
            row = cntall_v[sp, pl.ds(0, 16)]
            for h in range(2):
                j = 2 * sp + h
                tie_j = jnp.sum(jnp.where(iota == h, row, 0))
                quota_j = jnp.clip(rk - tie_acc, 0, tie_j)
                my_quota = jnp.where(j0 == j, quota_j, my_quota)
                tie_acc = tie_acc + tie_j
        cnt_v[pl.ds(0, 16)] = jnp.where(iota == 0, my_quota, 0)

    my_quota = jnp.sum(jnp.where(iota == 0, cnt_v[pl.ds(0, 16)], 0))

    base = j0 * 512
    Tm = T ^ _INT_MIN

    def sel_body(v, tiec):
        kb = key_v[pl.ds(c * 512 + v * 16, 16)]
        m = kb ^ _INT_MIN
        tie = kb == T
        ind = jnp.where(tie, 1, 0)
        rank = tiec + plsc.cumsum(ind) - 1
        sel = (m > Tm) | (tie & (rank < my_quota))
        maskb_v[pl.ds(v * 16, 16)] = jnp.where(sel, 1.0, 0.0)
        return tiec + jnp.sum(ind)
    lax.fori_loop(0, 32, sel_body, jnp.int32(0))

    pltpu.sync_copy(maskb_v,
                    mask_hbm.at[pl.ds(pl.multiple_of(base, 512), 512)])


def _select_sc(scores):
    mesh = plsc.VectorSubcoreMesh(core_axis_name="c", subcore_axis_name="s")
    call = pl.kernel(
        _select_body,
        out_type=jax.ShapeDtypeStruct((N,), jnp.float32),
        mesh=mesh,
        compiler_params=pltpu.CompilerParams(needs_layout_passes=False),
        scratch_types=[
            pltpu.VMEM((1024,), jnp.float32),
            pltpu.VMEM((1024,), jnp.int32),
            pltpu.VMEM((256,), jnp.int32),
            pltpu.VMEM((256,), jnp.int32),
            pltpu.VMEM((256,), jnp.int32),
            pltpu.VMEM((512,), jnp.float32),
            pltpu.VMEM((256,), jnp.int32),
            pltpu.VMEM((16, 256), jnp.int32),
            pltpu.VMEM_SHARED((1024,), jnp.int32),
            pltpu.VMEM_SHARED((16, 256), jnp.int32),
        ],
    )
    return call(scores)


def _mlp_body(x_ref, mt_ref, w1_ref, b1_ref, w2_ref, b2_ref, o_ref):
    xb = x_ref[...]
    dn = (((1,), (1,)), ((), ()))
    h = jax.lax.dot_general(xb, w1_ref[...], dn)
    h = jnp.maximum(h + b1_ref[...], 0.0)
    o = jax.lax.dot_general(h, w2_ref[...], dn) + b2_ref[...]
    mt = lax.transpose(mt_ref[...], (1, 0))
    for q in range(64):
        o_ref[128 * q:128 * (q + 1), :] = (
            o[128 * q:128 * (q + 1), :] * mt[:, q:q + 1])


def _mlp_tc(x, mask2d, W1, b1, W2, b2):
    return pl.pallas_call(
        _mlp_body,
        grid=(2,),
        in_specs=[
            pl.BlockSpec((8192, 128), lambda i: (i, 0)),
            pl.BlockSpec((64, 128), lambda i: (i, 0)),
            pl.BlockSpec((128, 128), lambda i: (0, 0)),
            pl.BlockSpec((1, 128), lambda i: (0, 0)),
            pl.BlockSpec((128, 128), lambda i: (0, 0)),
            pl.BlockSpec((1, 128), lambda i: (0, 0)),
        ],
        out_specs=pl.BlockSpec((8192, 128), lambda i: (i, 0)),
        out_shape=jax.ShapeDtypeStruct((N, D), jnp.float32),
    )(x, mask2d, W1, b1.reshape(1, D), W2, b2.reshape(1, D))


def kernel(x, W1, b1, W2, b2, Wg, bg):
    scores = _scores_tc(x, Wg, bg)
    mask = _select_sc(scores)
    out = _mlp_tc(x, mask.reshape(128, 128), W1, b1, W2, b2)
    return out, mask
